# Initial kernel scaffold; baseline (speedup 1.0000x reference)
import jax
import jax.numpy as jnp
from jax.experimental import pallas as pl


def kernel(x, edge_index, edge_weight, batch_ids, speed, W1, b1, W2, b2, Ws, bs, Wl1, bl1, Wl2, bl2):
    raise NotImplementedError("write your pallas kernel here")


# speedup vs baseline: 16.8857x; 16.8857x over previous
import jax
import jax.numpy as jnp
from jax import lax
from jax.experimental import pallas as pl
from jax.experimental.pallas import tpu as pltpu
from jax.experimental.pallas import tpu_sc as plsc

N = 10000
NP = 10240
E = 320000
D = 128
NC = 2
NS = 16
NW = NC * NS
EPW = E // NW
CHR = 125
CHE = 128
NCH = EPW // CHR

_mesh = plsc.VectorSubcoreMesh(core_axis_name="c", subcore_axis_name="s")

_GDN = lax.GatherDimensionNumbers(
    offset_dims=(), collapsed_slice_dims=(0,), start_index_map=(0,))


def _lane_bcast(vec, j):
    idx = jnp.full((16, 1), j, jnp.int32)
    return lax.gather(vec, idx, _GDN, (1,),
                      mode=lax.GatherScatterMode.PROMISE_IN_BOUNDS)


def _deg_body(dst_hbm, w_hbm, out_hbm, dst_v, w_v, zbuf, acc_sh):
    c = lax.axis_index("c")
    s = lax.axis_index("s")
    wid = c * NS + s

    def _z(i, carry):
        zbuf[pl.ds(i * 16, 16)] = jnp.zeros((16,), jnp.float32)
        return carry

    lax.fori_loop(0, 80, _z, 0)

    @pl.when(s < 8)
    def _():
        pltpu.sync_copy(zbuf, acc_sh.at[pl.ds(s * 1280, 1280)])

    plsc.subcore_barrier()

    pltpu.sync_copy(dst_hbm.at[wid], dst_v)
    pltpu.sync_copy(w_hbm.at[wid], w_v)

    def _chunk(ci, carry):
        pltpu.sync_copy(w_v.at[ci], acc_sh.at[dst_v.at[ci]], add=True)
        return carry

    lax.fori_loop(0, NCH, _chunk, 0)
    plsc.subcore_barrier()

    @pl.when(s < 8)
    def _():
        pltpu.sync_copy(acc_sh.at[pl.ds(s * 1280, 1280)],
                        out_hbm.at[pl.ds(c * NP + s * 1280, 1280)])


def _deg_call(dstp, wp):
    return pl.kernel(
        _deg_body,
        out_type=jax.ShapeDtypeStruct((NC * NP,), jnp.float32),
        mesh=_mesh,
        scratch_types=[
            pltpu.VMEM((NCH, CHE), jnp.int32),
            pltpu.VMEM((NCH, CHE), jnp.float32),
            pltpu.VMEM((1280,), jnp.float32),
            pltpu.VMEM_SHARED((NP,), jnp.float32),
        ],
    )(dstp, wp)


def _edge_body(src_hbm, dst_hbm, w_hbm, y_hbm, out_hbm,
               src_v, dst_v, w_v, rows_v, acc_sh):
    c = lax.axis_index("c")
    s = lax.axis_index("s")
    wid = c * NS + s

    for k in range(8):
        def _z(i, carry, k=k):
            rows_v[i, pl.ds(k * 16, 16)] = jnp.zeros((16,), jnp.float32)
            return carry
        lax.fori_loop(0, CHE, _z, 0)
    for j in range(5):
        pltpu.sync_copy(rows_v, acc_sh.at[pl.ds(s * 640 + j * 128, 128)])
    plsc.subcore_barrier()

    pltpu.sync_copy(src_hbm.at[wid], src_v)
    pltpu.sync_copy(dst_hbm.at[wid], dst_v)
    pltpu.sync_copy(w_hbm.at[wid], w_v)

    def _chunk(ci, carry):
        pltpu.sync_copy(y_hbm.at[src_v.at[ci]], rows_v)

        def _scale(g, c2):
            w16 = w_v[ci, pl.ds(g * 16, 16)]
            for j in range(16):
                wv = _lane_bcast(w16, j)
                e = g * 16 + j
                for k in range(8):
                    sl = pl.ds(k * 16, 16)
                    rows_v[e, sl] = rows_v[e, sl] * wv
            return c2

        lax.fori_loop(0, CHE // 16, _scale, 0)

        pltpu.sync_copy(rows_v, acc_sh.at[dst_v.at[ci]], add=True)
        return carry

    lax.fori_loop(0, NCH, _chunk, 0)
    plsc.subcore_barrier()

    for j in range(5):
        pltpu.sync_copy(acc_sh.at[pl.ds(s * 640 + j * 128, 128)], rows_v)
        pltpu.sync_copy(rows_v,
                        out_hbm.at[c, pl.ds(s * 640 + j * 128, 128)])


def _edge_call(srcp, dstp, wp, y):
    return pl.kernel(
        _edge_body,
        out_type=jax.ShapeDtypeStruct((NC, NP, D), jnp.float32),
        mesh=_mesh,
        scratch_types=[
            pltpu.VMEM((NCH, CHE), jnp.int32),
            pltpu.VMEM((NCH, CHE), jnp.int32),
            pltpu.VMEM((NCH, CHE), jnp.float32),
            pltpu.VMEM((CHE, D), jnp.float32),
            pltpu.VMEM_SHARED((NP, D), jnp.float32),
        ],
    )(srcp, dstp, wp, y)


def _tc1_body(degp_ref, x_ref, w1_ref, y1_ref):
    deg = degp_ref[0, :N] + degp_ref[1, :N] + 1.0
    dinv = lax.rsqrt(deg)
    xw = jnp.dot(x_ref[...], w1_ref[...], preferred_element_type=jnp.float32)
    y1_ref[...] = xw * dinv


def _tc2_body(degp_ref, p_ref, y_ref, b_ref, w_ref, out_ref):
    dinv = lax.rsqrt(degp_ref[0, :N] + degp_ref[1, :N] + 1.0)
    acc = p_ref[0, :N] + p_ref[1, :N] + y_ref[...]
    h = jnp.maximum(acc * dinv + b_ref[...], 0.0)
    out_ref[...] = jnp.dot(h, w_ref[...],
                           preferred_element_type=jnp.float32) * dinv


def _tc3_body(degp_ref, p_ref, y_ref, b_ref, bid_ref, speed_ref, ws_ref,
              bs_ref, wl1a_ref, wl1b_ref, bl1_ref, wl2_ref, bl2_ref, out_ref):
    dinv = lax.rsqrt(degp_ref[0, :N] + degp_ref[1, :N] + 1.0)
    h = jnp.maximum(
        (p_ref[0, :N] + p_ref[1, :N] + y_ref[...]) * dinv + b_ref[...],
        0.0)
    bid = bid_ref[...]
    neg = jnp.float32(-jnp.inf)
    gs = []
    for gid in range(16):
        m = bid == gid
        gs.append(jnp.max(jnp.where(m, h, neg), axis=0, keepdims=True))
    g = jnp.concatenate(gs, axis=0)
    v = speed_ref[...] * ws_ref[...] + bs_ref[...]
    vb = jnp.zeros((16, 16), jnp.float32)
    for kk in range(4):
        vb = vb + v[:, kk:kk + 1] * wl1b_ref[kk:kk + 1, :]
    hh = jnp.maximum(
        jnp.dot(g, wl1a_ref[...], preferred_element_type=jnp.float32)
        + vb + bl1_ref[...], 0.0)
    out_ref[...] = jnp.dot(hh, wl2_ref[...],
                           preferred_element_type=jnp.float32) + bl2_ref[...]


def _tc_call(body, out_shape, *args):
    return pl.pallas_call(
        body, out_shape=jax.ShapeDtypeStruct(out_shape, jnp.float32))(*args)


def kernel(x, edge_index, edge_weight, batch_ids, speed,
           W1, b1, W2, b2, Ws, bs, Wl1, bl1, Wl2, bl2):
    src = edge_index[0].astype(jnp.int32)
    dst = edge_index[1].astype(jnp.int32)
    ew = edge_weight.astype(jnp.float32)

    spread = (jnp.arange(NW * NCH * 3, dtype=jnp.int32) % N).reshape(
        NW * NCH, 3)

    def _shard_idx(a):
        a2 = a.reshape(NW * NCH, CHR)
        return jnp.concatenate([a2, spread], axis=1).reshape(NW, NCH, CHE)

    srcp = _shard_idx(src)
    dstp = _shard_idx(dst)
    wp = jnp.concatenate(
        [ew.reshape(NW * NCH, CHR),
         jnp.zeros((NW * NCH, 3), jnp.float32)], axis=1).reshape(NW, NCH, CHE)

    degp = _deg_call(dstp, wp).reshape(NC, NP, 1)

    y1 = _tc_call(_tc1_body, (N, D), degp, x, W1)
    P1 = _edge_call(srcp, dstp, wp, y1)
    y2 = _tc_call(_tc2_body, (N, D), degp, P1, y1, b1.reshape(1, D), W2)
    P2 = _edge_call(srcp, dstp, wp, y2)
    out = _tc_call(
        _tc3_body, (16, 16), degp, P2, y2, b2.reshape(1, D),
        batch_ids.astype(jnp.int32).reshape(N, 1), speed, Ws,
        bs.reshape(1, 4), Wl1[:D], Wl1[D:], bl1.reshape(1, 16), Wl2,
        bl2.reshape(1, 16))
    return out


# speedup vs baseline: 20.2025x; 1.1964x over previous
import jax
import jax.numpy as jnp
from jax import lax
from jax.experimental import pallas as pl
from jax.experimental.pallas import tpu as pltpu
from jax.experimental.pallas import tpu_sc as plsc

N = 10000
NP = 10240
E = 320000
D = 128
NC = 2
NS = 16
NW = NC * NS
CHE = 128
NCH = 80
EP = NW * NCH * CHE
NBUF = 2
NPH = 5
SLB = NCH // NPH

_mesh = plsc.VectorSubcoreMesh(core_axis_name="c", subcore_axis_name="s")

_GDN = lax.GatherDimensionNumbers(
    offset_dims=(), collapsed_slice_dims=(0,), start_index_map=(0,))


def _lane_bcast(vec, j):
    idx = jnp.full((16, 1), j, jnp.int32)
    return lax.gather(vec, idx, _GDN, (1,),
                      mode=lax.GatherScatterMode.PROMISE_IN_BOUNDS)


def _deg_body(dst_hbm, w_hbm, out_hbm, dst_v, w_v, zbuf, acc_sh):
    c = lax.axis_index("c")
    s = lax.axis_index("s")
    wid = c * NS + s

    def _z(i, carry):
        zbuf[pl.ds(i * 16, 16)] = jnp.zeros((16,), jnp.float32)
        return carry

    lax.fori_loop(0, 80, _z, 0)

    @pl.when(s < 8)
    def _():
        pltpu.sync_copy(zbuf, acc_sh.at[pl.ds(s * 1280, 1280)])

    plsc.subcore_barrier()

    pltpu.sync_copy(dst_hbm.at[wid], dst_v)
    pltpu.sync_copy(w_hbm.at[wid], w_v)

    def _chunk(ci, carry):
        pltpu.sync_copy(w_v.at[ci], acc_sh.at[dst_v.at[ci]], add=True)
        return carry

    lax.fori_loop(0, NCH, _chunk, 0)
    plsc.subcore_barrier()

    @pl.when(s < 8)
    def _():
        pltpu.sync_copy(acc_sh.at[pl.ds(s * 1280, 1280)],
                        out_hbm.at[pl.ds(c * NP + s * 1280, 1280)])


def _deg_call(dstp, wp):
    return pl.kernel(
        _deg_body,
        out_type=jax.ShapeDtypeStruct((NC * NP,), jnp.float32),
        mesh=_mesh,
        scratch_types=[
            pltpu.VMEM((NCH, CHE), jnp.int32),
            pltpu.VMEM((NCH, CHE), jnp.float32),
            pltpu.VMEM((1280,), jnp.float32),
            pltpu.VMEM_SHARED((NP,), jnp.float32),
        ],
    )(dstp, wp)


def _edge_body(src_hbm, dst_hbm, w_hbm, y_hbm, out_hbm,
               src_v, dst_v, w_v, rows_v, sg, ss, acc_sh):
    c = lax.axis_index("c")
    s = lax.axis_index("s")
    wid = c * NS + s

    for k in range(8):
        def _z(i, carry, k=k):
            rows_v[0, i, pl.ds(k * 16, 16)] = jnp.zeros((16,), jnp.float32)
            return carry
        lax.fori_loop(0, CHE, _z, 0)
    for j in range(5):
        pltpu.sync_copy(rows_v.at[0], acc_sh.at[pl.ds(s * 640 + j * 128, 128)])
    plsc.subcore_barrier()

    def _scale_chunk(ci, b):
        def _scale(g, c2):
            w16 = w_v[ci, pl.ds(g * 16, 16)]
            for j in range(16):
                wv = _lane_bcast(w16, j)
                for k in range(8):
                    sl = pl.ds(k * 16, 16)
                    rows_v[b, g * 16 + j, sl] = rows_v[b, g * 16 + j, sl] * wv
            return c2
        lax.fori_loop(0, CHE // 16, _scale, 0)

    NG = SLB // NBUF
    for p in range(NPH):
        pltpu.sync_copy(src_hbm.at[wid, pl.ds(p * SLB, SLB)], src_v)
        pltpu.sync_copy(dst_hbm.at[wid, pl.ds(p * SLB, SLB)], dst_v)
        pltpu.sync_copy(w_hbm.at[wid, pl.ds(p * SLB, SLB)], w_v)

        for b in range(NBUF):
            pltpu.async_copy(y_hbm.at[src_v.at[b]], rows_v.at[b], sg.at[b])

        def _group(g, carry):
            base = g * NBUF
            for b in range(NBUF):
                pltpu.make_async_copy(
                    y_hbm.at[pl.ds(0, CHE)], rows_v.at[b], sg.at[b]).wait()
                _scale_chunk(base + b, b)
                pltpu.async_copy(rows_v.at[b], acc_sh.at[dst_v.at[base + b]],
                                 ss.at[b], add=True)
            for b in range(NBUF):
                pltpu.make_async_copy(
                    rows_v.at[b], acc_sh.at[dst_v.at[base + b]],
                    ss.at[b]).wait()

            @pl.when(g + 1 < NG)
            def _():
                for b in range(NBUF):
                    pltpu.async_copy(y_hbm.at[src_v.at[base + NBUF + b]],
                                     rows_v.at[b], sg.at[b])
            return carry

        lax.fori_loop(0, NG, _group, 0)
    plsc.subcore_barrier()

    for j in range(5):
        pltpu.sync_copy(acc_sh.at[pl.ds(s * 640 + j * 128, 128)],
                        rows_v.at[0])
        pltpu.sync_copy(rows_v.at[0],
                        out_hbm.at[c, pl.ds(s * 640 + j * 128, 128)])


def _edge_call(srcp, dstp, wp, y):
    return pl.kernel(
        _edge_body,
        out_type=jax.ShapeDtypeStruct((NC, NP, D), jnp.float32),
        mesh=_mesh,
        scratch_types=[
            pltpu.VMEM((SLB, CHE), jnp.int32),
            pltpu.VMEM((SLB, CHE), jnp.int32),
            pltpu.VMEM((SLB, CHE), jnp.float32),
            pltpu.VMEM((NBUF, CHE, D), jnp.float32),
            pltpu.SemaphoreType.DMA((NBUF,)),
            pltpu.SemaphoreType.DMA((NBUF,)),
            pltpu.VMEM_SHARED((NP, D), jnp.float32),
        ],
    )(srcp, dstp, wp, y)


def _tc1_body(degp_ref, x_ref, w1_ref, y1_ref):
    deg = degp_ref[0, :N] + degp_ref[1, :N] + 1.0
    dinv = lax.rsqrt(deg)
    xw = jnp.dot(x_ref[...], w1_ref[...], preferred_element_type=jnp.float32)
    y1_ref[...] = xw * dinv


def _tc2_body(degp_ref, p_ref, y_ref, b_ref, w_ref, out_ref):
    dinv = lax.rsqrt(degp_ref[0, :N] + degp_ref[1, :N] + 1.0)
    acc = p_ref[0, :N] + p_ref[1, :N] + y_ref[...]
    h = jnp.maximum(acc * dinv + b_ref[...], 0.0)
    out_ref[...] = jnp.dot(h, w_ref[...],
                           preferred_element_type=jnp.float32) * dinv


def _tc3_body(degp_ref, p_ref, y_ref, b_ref, bid_ref, speed_ref, ws_ref,
              bs_ref, wl1a_ref, wl1b_ref, bl1_ref, wl2_ref, bl2_ref, out_ref):
    dinv = lax.rsqrt(degp_ref[0, :N] + degp_ref[1, :N] + 1.0)
    h = jnp.maximum(
        (p_ref[0, :N] + p_ref[1, :N] + y_ref[...]) * dinv + b_ref[...],
        0.0)
    bid = bid_ref[...]
    neg = jnp.float32(-jnp.inf)
    gs = []
    for gid in range(16):
        m = bid == gid
        gs.append(jnp.max(jnp.where(m, h, neg), axis=0, keepdims=True))
    g = jnp.concatenate(gs, axis=0)
    v = speed_ref[...] * ws_ref[...] + bs_ref[...]
    vb = jnp.zeros((16, 16), jnp.float32)
    for kk in range(4):
        vb = vb + v[:, kk:kk + 1] * wl1b_ref[kk:kk + 1, :]
    hh = jnp.maximum(
        jnp.dot(g, wl1a_ref[...], preferred_element_type=jnp.float32)
        + vb + bl1_ref[...], 0.0)
    out_ref[...] = jnp.dot(hh, wl2_ref[...],
                           preferred_element_type=jnp.float32) + bl2_ref[...]


def _tc_call(body, out_shape, *args):
    return pl.pallas_call(
        body, out_shape=jax.ShapeDtypeStruct(out_shape, jnp.float32))(*args)


def kernel(x, edge_index, edge_weight, batch_ids, speed,
           W1, b1, W2, b2, Ws, bs, Wl1, bl1, Wl2, bl2):
    src = edge_index[0].astype(jnp.int32)
    dst = edge_index[1].astype(jnp.int32)
    ew = edge_weight.astype(jnp.float32)

    npad = EP - E
    spread = jnp.arange(npad, dtype=jnp.int32) % N

    srcp = jnp.concatenate([src, spread]).reshape(NW, NCH, CHE)
    dstp = jnp.concatenate([dst, spread]).reshape(NW, NCH, CHE)
    wp = jnp.concatenate(
        [ew, jnp.zeros((npad,), jnp.float32)]).reshape(NW, NCH, CHE)

    degp = _deg_call(dstp, wp).reshape(NC, NP, 1)

    y1 = _tc_call(_tc1_body, (N, D), degp, x, W1)
    P1 = _edge_call(srcp, dstp, wp, y1)
    y2 = _tc_call(_tc2_body, (N, D), degp, P1, y1, b1.reshape(1, D), W2)
    P2 = _edge_call(srcp, dstp, wp, y2)
    out = _tc_call(
        _tc3_body, (16, 16), degp, P2, y2, b2.reshape(1, D),
        batch_ids.astype(jnp.int32).reshape(N, 1), speed, Ws,
        bs.reshape(1, 4), Wl1[:D], Wl1[D:], bl1.reshape(1, 16), Wl2,
        bl2.reshape(1, 16))
    return out
